# Initial kernel scaffold; baseline (speedup 1.0000x reference)
import jax
import jax.numpy as jnp
from jax.experimental import pallas as pl


def kernel(x, edge_index, batch, weight, w_ih, w_hh, b_ih, b_hh, local_W, local_b, global_W, global_b):
    raise NotImplementedError("write your pallas kernel here")


# speedup vs baseline: 6.3263x; 6.3263x over previous
import functools

import jax
import jax.numpy as jnp
from jax import lax
from jax.experimental import pallas as pl
from jax.experimental.pallas import tpu as pltpu
from jax.experimental.pallas import tpu_sc as plsc

N = 10000
E = 320000
H = 128
C = 10
L = 3
G = 64

NC = 2
NS = 16
EPT = E // NS
CH = 80
NCHUNK = EPT // CH
LANES = 16

NHALF = 5120
NTRASH = 128
NACC = NHALF + NTRASH
RPTZ = NACC // NS
RPTO = NHALF // NS
NPAD = 2 * NHALF

RB = 1000
NBLK = N // RB


_sc_mesh = plsc.VectorSubcoreMesh(core_axis_name="c", subcore_axis_name="s")


@functools.partial(
    pl.kernel,
    mesh=_sc_mesh,
    out_type=jax.ShapeDtypeStruct((NPAD, H), jnp.float32),
    scratch_types=[
        pltpu.VMEM((NCHUNK, CH), jnp.int32),
        pltpu.VMEM((NCHUNK, CH), jnp.int32),
        pltpu.VMEM((CH, H), jnp.float32),
        pltpu.VMEM((CH, H), jnp.float32),
        pltpu.VMEM_SHARED((NACC, H), jnp.float32),
        pltpu.SemaphoreType.DMA,
        pltpu.SemaphoreType.DMA,
    ],
)
def _sc_segment_sum(m_hbm, src_hbm, dst_hbm, zero_hbm, out_hbm,
                    src_v, dst_v, rows_a, rows_b, agg_s, sem_a, sem_b):
    c = lax.axis_index("c")
    s = lax.axis_index("s")

    pltpu.sync_copy(src_hbm.at[s], src_v)
    pltpu.sync_copy(dst_hbm.at[s], dst_v)
    pltpu.sync_copy(zero_hbm, agg_s.at[pl.ds(s * RPTZ, RPTZ)])

    lo = c * NHALF

    def remap_row(r, carry):
        for k in range(CH // LANES):
            d = dst_v[r, pl.ds(k * LANES, LANES)]
            off = d - lo
            inr = (off >= 0) & (off < NHALF)
            trash = NHALF + jnp.bitwise_and(d, NTRASH - 1)
            dst_v[r, pl.ds(k * LANES, LANES)] = jnp.where(inr, off, trash)
        return carry

    lax.fori_loop(0, NCHUNK, remap_row, 0)
    plsc.subcore_barrier()

    def gather_start(g, buf, sem):
        pltpu.async_copy(m_hbm.at[src_v.at[g]], buf, sem)

    def gather_wait(g, buf, sem):
        pltpu.make_async_copy(m_hbm.at[src_v.at[g]], buf, sem).wait()

    def scatter_add(g, buf):
        pltpu.sync_copy(buf, agg_s.at[dst_v.at[g]], add=True)

    gather_start(0, rows_a, sem_a)

    def body(i, carry):
        g = 2 * i
        gather_start(g + 1, rows_b, sem_b)
        gather_wait(g, rows_a, sem_a)
        scatter_add(g, rows_a)
        gather_start(g + 2, rows_a, sem_a)
        gather_wait(g + 1, rows_b, sem_b)
        scatter_add(g + 1, rows_b)
        return carry

    lax.fori_loop(0, NCHUNK // 2 - 1, body, 0)
    g_last = NCHUNK - 2
    gather_start(g_last + 1, rows_b, sem_b)
    gather_wait(g_last, rows_a, sem_a)
    scatter_add(g_last, rows_a)
    gather_wait(g_last + 1, rows_b, sem_b)
    scatter_add(g_last + 1, rows_b)

    plsc.subcore_barrier()
    pltpu.sync_copy(agg_s.at[pl.ds(s * RPTO, RPTO)],
                    out_hbm.at[pl.ds(c * NHALF + s * RPTO, RPTO)])


def _mm_body(x_ref, w_ref, m_ref):
    m_ref[...] = jnp.dot(x_ref[...], w_ref[...],
                         preferred_element_type=jnp.float32)


_mm_call = pl.pallas_call(
    _mm_body,
    grid=(NBLK,),
    in_specs=[
        pl.BlockSpec((RB, H), lambda i: (i, 0)),
        pl.BlockSpec((H, H), lambda i: (0, 0)),
    ],
    out_specs=pl.BlockSpec((RB, H), lambda i: (i, 0)),
    out_shape=jax.ShapeDtypeStruct((N, H), jnp.float32),
)


def _gru(h, agg, wih_ref, whh_ref, bih_ref, bhh_ref):
    gi = lax.dot_general(agg, wih_ref[...], (((1,), (1,)), ((), ())),
                         preferred_element_type=jnp.float32) + bih_ref[...]
    gh = lax.dot_general(h, whh_ref[...], (((1,), (1,)), ((), ())),
                         preferred_element_type=jnp.float32) + bhh_ref[...]
    r = jax.nn.sigmoid(gi[:, :H] + gh[:, :H])
    z = jax.nn.sigmoid(gi[:, H:2 * H] + gh[:, H:2 * H])
    n = jnp.tanh(gi[:, 2 * H:] + r * gh[:, 2 * H:])
    return (1.0 - z) * n + z * h


def _gru_mid_body(h_ref, a_ref, wih_ref, whh_ref, bih_ref, bhh_ref,
                  wn_ref, h_out, m_out):
    h_new = _gru(h_ref[...], a_ref[...], wih_ref, whh_ref, bih_ref, bhh_ref)
    h_out[...] = h_new
    m_out[...] = jnp.dot(h_new, wn_ref[...], preferred_element_type=jnp.float32)


_gru_mid_call = pl.pallas_call(
    _gru_mid_body,
    grid=(NBLK,),
    in_specs=[
        pl.BlockSpec((RB, H), lambda i: (i, 0)),
        pl.BlockSpec((RB, H), lambda i: (i, 0)),
        pl.BlockSpec((3 * H, H), lambda i: (0, 0)),
        pl.BlockSpec((3 * H, H), lambda i: (0, 0)),
        pl.BlockSpec((1, 3 * H), lambda i: (0, 0)),
        pl.BlockSpec((1, 3 * H), lambda i: (0, 0)),
        pl.BlockSpec((H, H), lambda i: (0, 0)),
    ],
    out_specs=[
        pl.BlockSpec((RB, H), lambda i: (i, 0)),
        pl.BlockSpec((RB, H), lambda i: (i, 0)),
    ],
    out_shape=[
        jax.ShapeDtypeStruct((N, H), jnp.float32),
        jax.ShapeDtypeStruct((N, H), jnp.float32),
    ],
)


def _final_body(h_ref, a_ref, wih_ref, whh_ref, bih_ref, bhh_ref,
                lw_ref, lb_ref, gw_ref, gb_ref, batch_ref,
                o_ref, sums_ref, cnt_ref):
    i = pl.program_id(0)

    @pl.when(i == 0)
    def _():
        sums_ref[...] = jnp.zeros_like(sums_ref)
        cnt_ref[...] = jnp.zeros_like(cnt_ref)

    h_new = _gru(h_ref[...], a_ref[...], wih_ref, whh_ref, bih_ref, bhh_ref)
    local = jax.nn.relu(
        lax.dot_general(h_new, lw_ref[...], (((1,), (1,)), ((), ())),
                        preferred_element_type=jnp.float32) + lb_ref[...])

    b = batch_ref[0, 0, :]
    onehot = (b[:, None] == lax.broadcasted_iota(jnp.int32, (RB, G), 1)
              ).astype(jnp.float32)
    sums_ref[...] += lax.dot_general(onehot, local, (((0,), (0,)), ((), ())),
                                     preferred_element_type=jnp.float32)
    cnt_ref[...] += lax.dot_general(onehot, jnp.ones((RB, H), jnp.float32),
                                    (((0,), (0,)), ((), ())),
                                    preferred_element_type=jnp.float32)

    @pl.when(i == NBLK - 1)
    def _():
        pooled = sums_ref[...] / jnp.maximum(cnt_ref[...], 1.0)
        logits = lax.dot_general(pooled, gw_ref[...], (((1,), (1,)), ((), ())),
                                 preferred_element_type=jnp.float32) + gb_ref[...]
        mx = jnp.max(logits, axis=-1, keepdims=True)
        lse = jnp.log(jnp.sum(jnp.exp(logits - mx), axis=-1, keepdims=True))
        o_ref[...] = logits - mx - lse


_final_call = pl.pallas_call(
    _final_body,
    grid=(NBLK,),
    in_specs=[
        pl.BlockSpec((RB, H), lambda i: (i, 0)),
        pl.BlockSpec((RB, H), lambda i: (i, 0)),
        pl.BlockSpec((3 * H, H), lambda i: (0, 0)),
        pl.BlockSpec((3 * H, H), lambda i: (0, 0)),
        pl.BlockSpec((1, 3 * H), lambda i: (0, 0)),
        pl.BlockSpec((1, 3 * H), lambda i: (0, 0)),
        pl.BlockSpec((H, H), lambda i: (0, 0)),
        pl.BlockSpec((1, H), lambda i: (0, 0)),
        pl.BlockSpec((C, H), lambda i: (0, 0)),
        pl.BlockSpec((1, C), lambda i: (0, 0)),
        pl.BlockSpec((1, 1, RB), lambda i: (i, 0, 0)),
    ],
    out_specs=pl.BlockSpec((G, C), lambda i: (0, 0)),
    out_shape=jax.ShapeDtypeStruct((G, C), jnp.float32),
    scratch_shapes=[
        pltpu.VMEM((G, H), jnp.float32),
        pltpu.VMEM((G, H), jnp.float32),
    ],
)


def kernel(x, edge_index, batch, weight, w_ih, w_hh, b_ih, b_hh,
           local_W, local_b, global_W, global_b):
    edge = edge_index.astype(jnp.int32)
    src3 = edge[0].reshape(NS, NCHUNK, CH)
    dst3 = edge[1].reshape(NS, NCHUNK, CH)
    batch3 = batch.astype(jnp.int32).reshape(NBLK, 1, RB)
    zeros = jnp.zeros((RPTZ, H), jnp.float32)
    bih2 = b_ih.reshape(1, 3 * H)
    bhh2 = b_hh.reshape(1, 3 * H)
    lb2 = local_b.reshape(1, H)
    gb2 = global_b.reshape(1, C)

    h = x
    m = _mm_call(x, weight[0])
    for i in range(L):
        agg = _sc_segment_sum(m, src3, dst3, zeros)
        if i < L - 1:
            h, m = _gru_mid_call(h, agg, w_ih, w_hh, bih2, bhh2,
                                 weight[i + 1])
        else:
            out = _final_call(h, agg, w_ih, w_hh, bih2, bhh2,
                              local_W, lb2, global_W, gb2, batch3)
    return out
